# Initial kernel scaffold; baseline (speedup 1.0000x reference)
import jax
import jax.numpy as jnp
from jax.experimental import pallas as pl


def kernel(x, edge_index, W1a, b1a, W1b, b1b, W2a, b2a, W2b, b2b):
    raise NotImplementedError("write your pallas kernel here")


# speedup vs baseline: 1.2121x; 1.2121x over previous
import functools

import jax
import jax.numpy as jnp
from jax import lax
from jax.experimental import pallas as pl
from jax.experimental.pallas import tpu as pltpu
from jax.experimental.pallas import tpu_sc as plsc

N_NODES = 10000
D = 256
EPS = 0.5
NC = 2
NS = 16
NW = NC * NS


def _mm_bias(x, W, b, block):
    N, K = x.shape
    M = W.shape[1]
    return pl.pallas_call(
        lambda xr, wr, br, outr: outr.__setitem__(
            ..., jnp.dot(xr[...], wr[...], preferred_element_type=jnp.float32)
            + br[...]),
        grid=(N // block,),
        in_specs=[
            pl.BlockSpec((block, K), lambda i: (i, 0)),
            pl.BlockSpec((K, M), lambda i: (0, 0)),
            pl.BlockSpec((1, M), lambda i: (0, 0)),
        ],
        out_specs=pl.BlockSpec((block, M), lambda i: (i, 0)),
        out_shape=jax.ShapeDtypeStruct((N, M), jnp.float32),
    )(x, W, b)


def _mid_chain(t, W1, b1, W2, b2, block):
    N, K = t.shape
    M = W2.shape[1]

    def body(tr, w1r, b1r, w2r, b2r, outr):
        h = jnp.maximum(tr[...], 0.0)
        h = jnp.dot(h, w1r[...], preferred_element_type=jnp.float32) + b1r[...]
        h = jnp.maximum(h, 0.0)
        outr[...] = (jnp.dot(h, w2r[...], preferred_element_type=jnp.float32)
                     + b2r[...])

    return pl.pallas_call(
        body,
        grid=(N // block,),
        in_specs=[
            pl.BlockSpec((block, K), lambda i: (i, 0)),
            pl.BlockSpec((K, W1.shape[1]), lambda i: (0, 0)),
            pl.BlockSpec((1, W1.shape[1]), lambda i: (0, 0)),
            pl.BlockSpec((W1.shape[1], M), lambda i: (0, 0)),
            pl.BlockSpec((1, M), lambda i: (0, 0)),
        ],
        out_specs=pl.BlockSpec((block, M), lambda i: (i, 0)),
        out_shape=jax.ShapeDtypeStruct((N, M), jnp.float32),
    )(t, W1, b1, W2, b2)


def _final_mm(u, W, b, block):
    N, K = u.shape
    M = W.shape[1]

    def body(ur, wr, br, outr):
        h = jnp.maximum(ur[...], 0.0)
        outr[...] = (jnp.dot(h, wr[...], preferred_element_type=jnp.float32)
                     + br[...])

    return pl.pallas_call(
        body,
        grid=(N // block,),
        in_specs=[
            pl.BlockSpec((block, K), lambda i: (i, 0)),
            pl.BlockSpec((K, M), lambda i: (0, 0)),
            pl.BlockSpec((1, M), lambda i: (0, 0)),
        ],
        out_specs=pl.BlockSpec((block, M), lambda i: (i, 0)),
        out_shape=jax.ShapeDtypeStruct((N, M), jnp.float32),
    )(u, W, b)


def _gather_add(table, src_idx, dst_idx, chunk):
    e_pad = src_idx.shape[0]
    per_w = e_pad // NW
    n_chunks = per_w // chunk
    mesh = plsc.VectorSubcoreMesh(core_axis_name="c", subcore_axis_name="s")

    @functools.partial(
        pl.kernel,
        out_type=jax.ShapeDtypeStruct((e_pad, D), jnp.float32),
        mesh=mesh,
        scratch_types=[
            pltpu.VMEM((chunk,), jnp.int32),
            pltpu.VMEM((chunk,), jnp.int32),
            pltpu.VMEM((chunk, D), jnp.float32),
            pltpu.VMEM((chunk, D), jnp.float32),
            pltpu.SemaphoreType.DMA,
            pltpu.SemaphoreType.DMA,
        ],
    )
    def k(table_hbm, src_hbm, dst_hbm, out_hbm, sidx, didx, bufa, bufb, sema, semb):
        wid = lax.axis_index("s") * NC + lax.axis_index("c")
        base = wid * per_w

        def chunk_body(c, carry):
            off = base + c * chunk
            pltpu.sync_copy(src_hbm.at[pl.ds(off, chunk)], sidx)
            pltpu.sync_copy(dst_hbm.at[pl.ds(off, chunk)], didx)
            cpa = pltpu.async_copy(table_hbm.at[sidx], bufa, sema)
            cpb = pltpu.async_copy(table_hbm.at[didx], bufb, semb)
            cpa.wait()
            cpb.wait()

            def add_row(r, carry2):
                def add16(j, carry3):
                    sl = pl.ds(j * 16, 16)
                    bufa[r, sl] = bufa[r, sl] + bufb[r, sl]
                    return carry3
                return lax.fori_loop(0, D // 16, add16, carry2, unroll=4)

            lax.fori_loop(0, chunk, add_row, 0)
            pltpu.sync_copy(bufa, out_hbm.at[pl.ds(off, chunk)])
            return carry

        lax.fori_loop(0, n_chunks, chunk_body, 0)

    return k(table, src_idx, dst_idx)


def _pad_idx(idx, e_pad):
    return jnp.concatenate(
        [idx, jnp.zeros((e_pad - idx.shape[0],), jnp.int32)])


def kernel(x, edge_index, W1a, b1a, W1b, b1b, W2a, b2a, W2b, b2b):
    n_edges = edge_index.shape[1]
    src = edge_index[0].astype(jnp.int32)
    dst = edge_index[1].astype(jnp.int32)

    e1_pad = 10240
    e2_pad = 163840
    src1 = _pad_idx(src[:N_NODES], e1_pad)
    dst1 = _pad_idx(dst[:N_NODES], e1_pad)
    src2 = _pad_idx(src, e2_pad)
    dst2 = _pad_idx(dst, e2_pad)

    q = _mm_bias(x, W1a, (0.5 * b1a)[None, :], block=1000)
    t = _gather_add(q, src1, dst1, chunk=80)
    p = _mid_chain(t, EPS * W1b, (EPS * b1b)[None, :],
                   W2a, (0.5 * b2a)[None, :], block=1024)
    u = _gather_add(p, src2, dst2, chunk=128)
    out = _final_mm(u, EPS * W2b, (EPS * b2b)[None, :], block=2048)
    return out[:n_edges]


# speedup vs baseline: 3.1874x; 2.6297x over previous
import functools

import jax
import jax.numpy as jnp
from jax import lax
from jax.experimental import pallas as pl
from jax.experimental.pallas import tpu as pltpu
from jax.experimental.pallas import tpu_sc as plsc

N_NODES = 10000
D = 256
EPS = 0.5
NC = 2
NS = 16
NW = NC * NS


def _mm_bias(x, W, b, block):
    N, K = x.shape
    M = W.shape[1]
    return pl.pallas_call(
        lambda xr, wr, br, outr: outr.__setitem__(
            ..., jnp.dot(xr[...], wr[...], preferred_element_type=jnp.float32)
            + br[...]),
        grid=(N // block,),
        in_specs=[
            pl.BlockSpec((block, K), lambda i: (i, 0)),
            pl.BlockSpec((K, M), lambda i: (0, 0)),
            pl.BlockSpec((1, M), lambda i: (0, 0)),
        ],
        out_specs=pl.BlockSpec((block, M), lambda i: (i, 0)),
        out_shape=jax.ShapeDtypeStruct((N, M), jnp.float32),
    )(x, W, b)


def _mid_chain(t, W1, b1, W2, b2, block):
    N, K = t.shape
    M = W2.shape[1]

    def body(tr, w1r, b1r, w2r, b2r, outr):
        h = jnp.maximum(tr[...], 0.0)
        h = jnp.dot(h, w1r[...], preferred_element_type=jnp.float32) + b1r[...]
        h = jnp.maximum(h, 0.0)
        outr[...] = (jnp.dot(h, w2r[...], preferred_element_type=jnp.float32)
                     + b2r[...])

    return pl.pallas_call(
        body,
        grid=(N // block,),
        in_specs=[
            pl.BlockSpec((block, K), lambda i: (i, 0)),
            pl.BlockSpec((K, W1.shape[1]), lambda i: (0, 0)),
            pl.BlockSpec((1, W1.shape[1]), lambda i: (0, 0)),
            pl.BlockSpec((W1.shape[1], M), lambda i: (0, 0)),
            pl.BlockSpec((1, M), lambda i: (0, 0)),
        ],
        out_specs=pl.BlockSpec((block, M), lambda i: (i, 0)),
        out_shape=jax.ShapeDtypeStruct((N, M), jnp.float32),
    )(t, W1, b1, W2, b2)


def _final_mm(u, W, b, block):
    N, K = u.shape
    M = W.shape[1]

    def body(ur, wr, br, outr):
        h = jnp.maximum(ur[...], 0.0)
        outr[...] = (jnp.dot(h, wr[...], preferred_element_type=jnp.float32)
                     + br[...])

    return pl.pallas_call(
        body,
        grid=(N // block,),
        in_specs=[
            pl.BlockSpec((block, K), lambda i: (i, 0)),
            pl.BlockSpec((K, M), lambda i: (0, 0)),
            pl.BlockSpec((1, M), lambda i: (0, 0)),
        ],
        out_specs=pl.BlockSpec((block, M), lambda i: (i, 0)),
        out_shape=jax.ShapeDtypeStruct((N, M), jnp.float32),
    )(u, W, b)


def _gather_add(table, src, dst, n_edges, chunk, n_main, tail_chunk, n_tail):
    per_w = n_main * chunk
    tail_base = NW * per_w
    mesh = plsc.VectorSubcoreMesh(core_axis_name="c", subcore_axis_name="s")

    @functools.partial(
        pl.kernel,
        out_type=jax.ShapeDtypeStruct((n_edges, D), jnp.float32),
        mesh=mesh,
        scratch_types=[
            pltpu.VMEM((per_w,), jnp.int32),
            pltpu.VMEM((per_w,), jnp.int32),
            pltpu.VMEM((tail_chunk,), jnp.int32),
            pltpu.VMEM((tail_chunk,), jnp.int32),
            pltpu.VMEM((chunk, D), jnp.float32),
            pltpu.VMEM((chunk, D), jnp.float32),
            pltpu.VMEM((chunk, D), jnp.float32),
            pltpu.VMEM((chunk, D), jnp.float32),
            pltpu.SemaphoreType.DMA,
            pltpu.SemaphoreType.DMA,
            pltpu.SemaphoreType.DMA,
            pltpu.SemaphoreType.DMA,
            pltpu.SemaphoreType.DMA,
            pltpu.SemaphoreType.DMA,
        ],
    )
    def k(table_hbm, src_hbm, dst_hbm, out_hbm, sidx, didx, tsidx, tdidx,
          rowsa0, rowsa1, rowsb0, rowsb1, ga0, ga1, gb0, gb1, ws0, ws1):
        wid = lax.axis_index("s") * NC + lax.axis_index("c")
        base = wid * per_w
        rowsa = (rowsa0, rowsa1)
        rowsb = (rowsb0, rowsb1)
        ga = (ga0, ga1)
        gb = (gb0, gb1)
        ws = (ws0, ws1)

        def add_rows(ba, bb, n_rows):
            def add_row(r, carry):
                for j in range(D // 16):
                    sl = pl.ds(j * 16, 16)
                    ba[r, sl] = ba[r, sl] + bb[r, sl]
                return carry
            lax.fori_loop(0, n_rows, add_row, 0)

        pltpu.sync_copy(src_hbm.at[pl.ds(base, per_w)], sidx)
        pltpu.sync_copy(dst_hbm.at[pl.ds(base, per_w)], didx)

        @pl.when(wid < n_tail)
        def _():
            toff = tail_base + wid * tail_chunk
            pltpu.sync_copy(src_hbm.at[pl.ds(toff, tail_chunk)], tsidx)
            pltpu.sync_copy(dst_hbm.at[pl.ds(toff, tail_chunk)], tdidx)
            tra = rowsa0.at[pl.ds(0, tail_chunk)]
            trb = rowsb0.at[pl.ds(0, tail_chunk)]
            pltpu.async_copy(table_hbm.at[tsidx], tra, ga0)
            pltpu.async_copy(table_hbm.at[tdidx], trb, gb0)
            pltpu.make_async_copy(
                table_hbm.at[pl.ds(0, tail_chunk)], tra, ga0).wait()
            pltpu.make_async_copy(
                table_hbm.at[pl.ds(0, tail_chunk)], trb, gb0).wait()
            add_rows(rowsa0, rowsb0, tail_chunk)
            pltpu.sync_copy(tra, out_hbm.at[pl.ds(toff, tail_chunk)])

        pltpu.async_copy(table_hbm.at[sidx.at[pl.ds(0, chunk)]], rowsa0, ga0)
        pltpu.async_copy(table_hbm.at[didx.at[pl.ds(0, chunk)]], rowsb0, gb0)

        def step(c, b, nb):
            pltpu.make_async_copy(
                table_hbm.at[pl.ds(0, chunk)], rowsa[b], ga[b]).wait()
            pltpu.make_async_copy(
                table_hbm.at[pl.ds(0, chunk)], rowsb[b], gb[b]).wait()
            @pl.when(c + 1 < n_main)
            def _():
                @pl.when(c >= 1)
                def _():
                    pltpu.make_async_copy(
                        rowsa[nb], out_hbm.at[pl.ds(0, chunk)], ws[nb]).wait()
                pltpu.async_copy(
                    table_hbm.at[sidx.at[pl.ds((c + 1) * chunk, chunk)]],
                    rowsa[nb], ga[nb])
                pltpu.async_copy(
                    table_hbm.at[didx.at[pl.ds((c + 1) * chunk, chunk)]],
                    rowsb[nb], gb[nb])
            add_rows(rowsa[b], rowsb[b], chunk)
            pltpu.async_copy(
                rowsa[b], out_hbm.at[pl.ds(base + c * chunk, chunk)], ws[b])

        def pair(c2, carry):
            c = c2 * 2
            step(c, 0, 1)
            @pl.when(c + 1 < n_main)
            def _():
                step(c + 1, 1, 0)
            return carry

        lax.fori_loop(0, (n_main + 1) // 2, pair, 0)

        pltpu.make_async_copy(rowsa0, out_hbm.at[pl.ds(0, chunk)], ws0).wait()
        pltpu.make_async_copy(rowsa1, out_hbm.at[pl.ds(0, chunk)], ws1).wait()

    return k(table, src, dst)


def kernel(x, edge_index, W1a, b1a, W1b, b1b, W2a, b2a, W2b, b2b):
    n_edges = edge_index.shape[1]
    src = edge_index[0].astype(jnp.int32)
    dst = edge_index[1].astype(jnp.int32)

    q = _mm_bias(x, W1a, (0.5 * b1a)[None, :], block=1000)
    t = _gather_add(q, src, dst, N_NODES,
                    chunk=104, n_main=3, tail_chunk=16, n_tail=1)
    p = _mid_chain(t, EPS * W1b, (EPS * b1b)[None, :],
                   W2a, (0.5 * b2a)[None, :], block=1000)
    u = _gather_add(p, src, dst, n_edges,
                    chunk=104, n_main=48, tail_chunk=64, n_tail=4)
    return _final_mm(u, EPS * W2b, (EPS * b2b)[None, :], block=2000)
